# Initial kernel scaffold; baseline (speedup 1.0000x reference)
import jax
import jax.numpy as jnp
from jax.experimental import pallas as pl


def kernel(xs_F_cat, edge_vec, W0, Wr1a, Wr1b, W_out, edge_idx, atomic_numbers):
    raise NotImplementedError("write your pallas kernel here")


# speedup vs baseline: 3.6982x; 3.6982x over previous
import functools

import jax
import jax.numpy as jnp
from jax import lax
from jax.experimental import pallas as pl
from jax.experimental.pallas import tpu as pltpu
from jax.experimental.pallas import tpu_sc as plsc

E = 320000
D_IN = 512
EMB = 128
N_AT = 10000

BLK = 2560
GRID = E // BLK

NC = 2
NS = 16
NW = NC * NS
EP = 327680
CHUNK = EP // NW
ROWS = CHUNK // 128
NP = 10240
NPR = NP // 128

_INV_SQRT2 = 0.7071067811865476


def _ssilu(x):
    return (x * jax.nn.sigmoid(x)) * (1.0 / 0.6)


def _mlp_body(xs_ref, evx_ref, evy_ref, evz_ref, w0_ref, wa_ref, wb_ref,
              wo_ref, vx_ref, vy_ref, vz_ref):
    x = jnp.dot(xs_ref[...], w0_ref[...], preferred_element_type=jnp.float32)
    x = _ssilu(x)
    h = _ssilu(jnp.dot(x, wa_ref[...], preferred_element_type=jnp.float32))
    h = _ssilu(jnp.dot(h, wb_ref[...], preferred_element_type=jnp.float32))
    x = (x + h) * _INV_SQRT2
    xt = jnp.transpose(x)
    ft = jnp.dot(wo_ref[...], xt, preferred_element_type=jnp.float32)
    vx_ref[...] = ft * evx_ref[...]
    vy_ref[...] = ft * evy_ref[...]
    vz_ref[...] = ft * evz_ref[...]


def _mlp(xs, evxt, evyt, evzt, W0, Wr1a, Wr1b, WoT):
    row = jax.ShapeDtypeStruct((1, E), jnp.float32)
    return pl.pallas_call(
        _mlp_body,
        grid=(GRID,),
        in_specs=[
            pl.BlockSpec((BLK, D_IN), lambda i: (i, 0)),
            pl.BlockSpec((1, BLK), lambda i: (0, i)),
            pl.BlockSpec((1, BLK), lambda i: (0, i)),
            pl.BlockSpec((1, BLK), lambda i: (0, i)),
            pl.BlockSpec((D_IN, EMB), lambda i: (0, 0)),
            pl.BlockSpec((EMB, EMB), lambda i: (0, 0)),
            pl.BlockSpec((EMB, EMB), lambda i: (0, 0)),
            pl.BlockSpec((1, EMB), lambda i: (0, 0)),
        ],
        out_specs=[
            pl.BlockSpec((1, BLK), lambda i: (0, i)),
            pl.BlockSpec((1, BLK), lambda i: (0, i)),
            pl.BlockSpec((1, BLK), lambda i: (0, i)),
        ],
        out_shape=[row, row, row],
    )(xs, evxt, evyt, evzt, W0, Wr1a, Wr1b, WoT)


ACC = 3 * NP


def _scatter_body(idx_hbm, vx_hbm, vy_hbm, vz_hbm, out_hbm,
                  ix_v, vx_v, vy_v, vz_v, acc):
    cid = lax.axis_index("c")
    sid = lax.axis_index("s")
    wid = sid * NC + cid
    r0 = wid * ROWS

    pltpu.sync_copy(idx_hbm.at[pl.ds(r0, ROWS)], ix_v)
    pltpu.sync_copy(vx_hbm.at[pl.ds(r0, ROWS)], vx_v)
    pltpu.sync_copy(vy_hbm.at[pl.ds(r0, ROWS)], vy_v)
    pltpu.sync_copy(vz_hbm.at[pl.ds(r0, ROWS)], vz_v)

    zero16 = jnp.zeros((16,), jnp.float32)

    def zrow(i, carry):
        acc[pl.ds(i * 16, 16)] = zero16
        return carry

    lax.fori_loop(0, ACC // 16, zrow, 0)

    def srow(r, carry):
        for c in range(8):
            sl = pl.ds(c * 16, 16)
            tgt = ix_v[r, sl]
            plsc.addupdate_scatter(acc, [tgt], vx_v[r, sl])
            plsc.addupdate_scatter(acc, [tgt + NP], vy_v[r, sl])
            plsc.addupdate_scatter(acc, [tgt + 2 * NP], vz_v[r, sl])
        return carry

    lax.fori_loop(0, ROWS, srow, 0)

    pltpu.sync_copy(acc, out_hbm.at[wid])


@functools.lru_cache(maxsize=1)
def _scatter_fn():
    return pl.kernel(
        _scatter_body,
        out_type=jax.ShapeDtypeStruct((NW, ACC), jnp.float32),
        mesh=plsc.VectorSubcoreMesh(core_axis_name="c", subcore_axis_name="s"),
        compiler_params=pltpu.CompilerParams(needs_layout_passes=False),
        scratch_types=[
            pltpu.VMEM((ROWS, 128), jnp.int32),
            pltpu.VMEM((ROWS, 128), jnp.float32),
            pltpu.VMEM((ROWS, 128), jnp.float32),
            pltpu.VMEM((ROWS, 128), jnp.float32),
            pltpu.VMEM((ACC,), jnp.float32),
        ],
    )


def _combine_body(p_ref, o_ref):
    o_ref[...] = jnp.sum(p_ref[...], axis=0, keepdims=True)


def _combine(partials):
    return pl.pallas_call(
        _combine_body,
        out_shape=jax.ShapeDtypeStruct((1, ACC), jnp.float32),
    )(partials)


def kernel(xs_F_cat, edge_vec, W0, Wr1a, Wr1b, W_out, edge_idx, atomic_numbers):
    evt = jnp.transpose(edge_vec)
    evxt = evt[0:1]
    evyt = evt[1:2]
    evzt = evt[2:3]
    vx, vy, vz = _mlp(xs_F_cat, evxt, evyt, evzt, W0, Wr1a, Wr1b,
                      jnp.transpose(W_out))
    pad = EP - E

    def to_rows(a):
        return jnp.pad(a.reshape(E), (0, pad)).reshape(EP // 128, 128)

    idx2 = jnp.pad(edge_idx.astype(jnp.int32), (0, pad)).reshape(EP // 128, 128)
    partials = _scatter_fn()(idx2, to_rows(vx), to_rows(vy), to_rows(vz))
    comb = _combine(partials)
    forces = jnp.transpose(comb.reshape(3, NP))
    return forces[:N_AT]
